# baseline (speedup 1.0000x reference)
import functools

import jax
import jax.numpy as jnp
from jax import lax
from jax.experimental import pallas as pl
from jax.experimental.pallas import tpu as pltpu
from jax.experimental.pallas import tpu_sc as plsc

B = 16384
D = 64
PD = 2 * D
V = 100000
_NC = 2
_NS = 16
NW = _NC * _NS
BPW = B // NW
CHUNK = 64
NCHUNK = BPW // CHUNK
NBUF = 8
BN = 8192


def _build_body(mfu_r, mlu_r, mfi_r, mli_r, wu_r, wi_r, wd_r, eye_r,
                ou_r, oi_r):
    dn = (((0,), (0,)), ((), ()))
    bf = jnp.bfloat16

    def mm(x, w):
        return lax.dot_general(x[...].astype(bf), w[...].astype(bf), dn,
                               preferred_element_type=jnp.float32)

    ou_r[:, :D] = mm(mfu_r, wd_r)
    ou_r[:, D:] = mm(mlu_r, wu_r)
    oi_r[:, :D] = mm(mfi_r, eye_r)
    oi_r[:, D:] = mm(mli_r, wi_r)


def _build_staged(mf_u, mlp_u, mf_i, mlp_i, W1, Wp):
    w1u = W1[:D]
    w1i = W1[D:]
    wdiag = jnp.diag(Wp[32:, 0])
    eye = jnp.eye(D, dtype=jnp.float32)
    grid = (pl.cdiv(V, BN),)
    tspec = pl.BlockSpec((D, BN), lambda i: (0, i))
    wspec = pl.BlockSpec((D, D), lambda i: (0, 0))
    out_spec = pl.BlockSpec((BN, PD), lambda i: (i, 0))
    return pl.pallas_call(
        _build_body,
        grid=grid,
        in_specs=[tspec, tspec, tspec, tspec, wspec, wspec, wspec, wspec],
        out_specs=[out_spec, out_spec],
        out_shape=[jax.ShapeDtypeStruct((V, PD), jnp.float32)] * 2,
    )(mf_u.T, mlp_u.T, mf_i.T, mlp_i.T, w1u, w1i, wdiag, eye)


def _sc_gather(uidx, iidx, ustage, istage, hb, hbpw, hnchunk):
    mesh = plsc.VectorSubcoreMesh(core_axis_name="c", subcore_axis_name="s")
    out_t = [jax.ShapeDtypeStruct((hb, PD), jnp.float32)] * 2
    scratch = [
        pltpu.VMEM((hnchunk, CHUNK), jnp.int32),
        pltpu.VMEM((hnchunk, CHUNK), jnp.int32),
    ] + [pltpu.VMEM((CHUNK, PD), jnp.float32) for _ in range(NBUF)] + [
        pltpu.SemaphoreType.DMA for _ in range(NBUF)
    ]

    @functools.partial(pl.kernel, mesh=mesh, out_type=out_t,
                       scratch_types=scratch)
    def k(uidx_h, iidx_h, us_h, is_h, ou, oi, uv, iv, *bufs_sems):
        bufs = bufs_sems[:NBUF]
        sems = bufs_sems[NBUF:]
        wid = lax.axis_index("s") * _NC + lax.axis_index("c")
        base = wid * hbpw
        pltpu.sync_copy(uidx_h.at[wid], uv)
        pltpu.sync_copy(iidx_h.at[wid], iv)
        sched = [(us_h, uv, ou, j) for j in range(hnchunk)] + \
                [(is_h, iv, oi, j) for j in range(hnchunk)]
        cps = [None] * len(sched)
        for k_ in range(len(sched)):
            b = k_ % NBUF
            if k_ >= NBUF:
                cps[k_ - NBUF].wait()
                _, _, po, pj = sched[k_ - NBUF]
                pltpu.sync_copy(bufs[b],
                                po.at[pl.ds(base + pj * CHUNK, CHUNK)])
            tbl, idxv, _, j = sched[k_]
            cps[k_] = pltpu.async_copy(tbl.at[idxv.at[j]], bufs[b], sems[b])
        for k_ in range(len(sched) - NBUF, len(sched)):
            b = k_ % NBUF
            cps[k_].wait()
            _, _, po, pj = sched[k_]
            pltpu.sync_copy(bufs[b], po.at[pl.ds(base + pj * CHUNK, CHUNK)])

    return k(uidx, iidx, ustage, istage)


BB = 8192


def _tc_body(u_r, i_r, b1_r, w2_r, b2_r, wpm_r, ones_r, bp_r, o_r):
    u = u_r[...]
    it = i_r[...]
    h1 = jnp.maximum(u[:, D:] + it[:, D:] + b1_r[...], 0.0)
    h2 = jnp.dot(h1, w2_r[...], preferred_element_type=jnp.float32)
    h2 = jnp.maximum(h2 + b2_r[...], 0.0)
    mf = u[:, :D] * it[:, :D]
    s1 = jnp.dot(h2, wpm_r[...], preferred_element_type=jnp.float32)
    s2 = jnp.dot(mf, ones_r[...], preferred_element_type=jnp.float32)
    logit = s1[:, 0] + s2[:, 0] + bp_r[0, 0]
    o_r[...] = jax.nn.sigmoid(logit)[None, :]


def _tc_dense(u_rows, i_rows, b1, W2, b2, Wp, bp, hb):
    b1r = b1.reshape(1, -1)
    b2r = b2.reshape(1, -1)
    wpm = Wp[:32].reshape(32, 1)
    ones = jnp.ones((D, 1), dtype=jnp.float32)
    bpr = bp.reshape(1, 1)

    row_spec = pl.BlockSpec((BB, PD), lambda i: (i, 0))
    full = lambda shape: pl.BlockSpec(shape, lambda i: (0, 0))
    out = pl.pallas_call(
        _tc_body,
        grid=(hb // BB,),
        in_specs=[
            row_spec, row_spec,
            full((1, 64)), full((64, 32)), full((1, 32)),
            full((32, 1)), full((D, 1)), full((1, 1)),
        ],
        out_specs=pl.BlockSpec((1, BB), lambda i: (0, i)),
        out_shape=jax.ShapeDtypeStruct((1, hb), jnp.float32),
    )(u_rows, i_rows, b1r, W2, b2r, wpm, ones, bpr)
    return out.reshape(-1)


def kernel(user_input, item_input, mf_user_table, mf_item_table,
           mlp_user_table, mlp_item_table, W1, b1, W2, b2, Wp, bp):
    u = user_input.astype(jnp.int32).reshape(NW, NCHUNK, CHUNK)
    it = item_input.astype(jnp.int32).reshape(NW, NCHUNK, CHUNK)
    ustage, istage = _build_staged(mf_user_table, mlp_user_table,
                                   mf_item_table, mlp_item_table, W1, Wp)
    u_rows, i_rows = _sc_gather(u, it, ustage, istage, B, BPW, NCHUNK)
    return _tc_dense(u_rows, i_rows, b1, W2, b2, Wp, bp, B)


# speedup vs baseline: 1.0413x; 1.0413x over previous
import functools

import jax
import jax.numpy as jnp
from jax import lax
from jax.experimental import pallas as pl
from jax.experimental.pallas import tpu as pltpu
from jax.experimental.pallas import tpu_sc as plsc

B = 16384
D = 64
PD = 2 * D
V = 100000
_NC = 2
_NS = 16
NW = _NC * _NS
BPW = B // NW
CHUNK = 128
NCHUNK = BPW // CHUNK
NBUF = 6
BN = 8192


def _build_body(mfu_r, mlu_r, mfi_r, mli_r, wu_r, wi_r, wd_r, eye_r,
                ou_r, oi_r):
    dn = (((0,), (0,)), ((), ()))
    bf = jnp.bfloat16

    def mm(x, w):
        return lax.dot_general(x[...].astype(bf), w[...].astype(bf), dn,
                               preferred_element_type=jnp.float32)

    ou_r[:, :D] = mm(mfu_r, wd_r)
    ou_r[:, D:] = mm(mlu_r, wu_r)
    oi_r[:, :D] = mm(mfi_r, eye_r)
    oi_r[:, D:] = mm(mli_r, wi_r)


def _build_staged(mf_u, mlp_u, mf_i, mlp_i, W1, Wp):
    w1u = W1[:D]
    w1i = W1[D:]
    wdiag = jnp.diag(Wp[32:, 0])
    eye = jnp.eye(D, dtype=jnp.float32)
    grid = (pl.cdiv(V, BN),)
    tspec = pl.BlockSpec((D, BN), lambda i: (0, i))
    wspec = pl.BlockSpec((D, D), lambda i: (0, 0))
    out_spec = pl.BlockSpec((BN, PD), lambda i: (i, 0))
    return pl.pallas_call(
        _build_body,
        grid=grid,
        in_specs=[tspec, tspec, tspec, tspec, wspec, wspec, wspec, wspec],
        out_specs=[out_spec, out_spec],
        out_shape=[jax.ShapeDtypeStruct((V, PD), jnp.float32)] * 2,
    )(mf_u.T, mlp_u.T, mf_i.T, mlp_i.T, w1u, w1i, wdiag, eye)


def _sc_gather(uidx, iidx, ustage, istage, hb, hbpw, hnchunk):
    mesh = plsc.VectorSubcoreMesh(core_axis_name="c", subcore_axis_name="s")
    out_t = [jax.ShapeDtypeStruct((hb, PD), jnp.float32)] * 2
    scratch = [
        pltpu.VMEM((hnchunk, CHUNK), jnp.int32),
        pltpu.VMEM((hnchunk, CHUNK), jnp.int32),
    ] + [pltpu.VMEM((CHUNK, PD), jnp.float32) for _ in range(NBUF)] + [
        pltpu.SemaphoreType.DMA for _ in range(NBUF)
    ]

    @functools.partial(pl.kernel, mesh=mesh, out_type=out_t,
                       scratch_types=scratch)
    def k(uidx_h, iidx_h, us_h, is_h, ou, oi, uv, iv, *bufs_sems):
        bufs = bufs_sems[:NBUF]
        sems = bufs_sems[NBUF:]
        wid = lax.axis_index("s") * _NC + lax.axis_index("c")
        base = wid * hbpw
        pltpu.sync_copy(uidx_h.at[wid], uv)
        pltpu.sync_copy(iidx_h.at[wid], iv)
        sched = [(us_h, uv, ou, j) for j in range(hnchunk)] + \
                [(is_h, iv, oi, j) for j in range(hnchunk)]
        cps = [None] * len(sched)
        for k_ in range(len(sched)):
            b = k_ % NBUF
            if k_ >= NBUF:
                cps[k_ - NBUF].wait()
                _, _, po, pj = sched[k_ - NBUF]
                pltpu.sync_copy(bufs[b],
                                po.at[pl.ds(base + pj * CHUNK, CHUNK)])
            tbl, idxv, _, j = sched[k_]
            cps[k_] = pltpu.async_copy(tbl.at[idxv.at[j]], bufs[b], sems[b])
        for k_ in range(len(sched) - NBUF, len(sched)):
            b = k_ % NBUF
            cps[k_].wait()
            _, _, po, pj = sched[k_]
            pltpu.sync_copy(bufs[b], po.at[pl.ds(base + pj * CHUNK, CHUNK)])

    return k(uidx, iidx, ustage, istage)


BB = 4096


def _tc_body(u_r, i_r, b1_r, w2_r, b2_r, wpm_r, ones_r, bp_r, o_r):
    u = u_r[...]
    it = i_r[...]
    h1 = jnp.maximum(u[:, D:] + it[:, D:] + b1_r[...], 0.0)
    h2 = jnp.dot(h1, w2_r[...], preferred_element_type=jnp.float32)
    h2 = jnp.maximum(h2 + b2_r[...], 0.0)
    mf = u[:, :D] * it[:, :D]
    s1 = jnp.dot(h2, wpm_r[...], preferred_element_type=jnp.float32)
    s2 = jnp.dot(mf, ones_r[...], preferred_element_type=jnp.float32)
    logit = s1[:, 0] + s2[:, 0] + bp_r[0, 0]
    o_r[...] = jax.nn.sigmoid(logit)[None, :]


def _tc_dense(u_rows, i_rows, b1, W2, b2, Wp, bp, hb):
    b1r = b1.reshape(1, -1)
    b2r = b2.reshape(1, -1)
    wpm = Wp[:32].reshape(32, 1)
    ones = jnp.ones((D, 1), dtype=jnp.float32)
    bpr = bp.reshape(1, 1)

    row_spec = pl.BlockSpec((BB, PD), lambda i: (i, 0))
    full = lambda shape: pl.BlockSpec(shape, lambda i: (0, 0))
    out = pl.pallas_call(
        _tc_body,
        grid=(hb // BB,),
        in_specs=[
            row_spec, row_spec,
            full((1, 64)), full((64, 32)), full((1, 32)),
            full((32, 1)), full((D, 1)), full((1, 1)),
        ],
        out_specs=pl.BlockSpec((1, BB), lambda i: (0, i)),
        out_shape=jax.ShapeDtypeStruct((1, hb), jnp.float32),
    )(u_rows, i_rows, b1r, W2, b2r, wpm, ones, bpr)
    return out.reshape(-1)


def kernel(user_input, item_input, mf_user_table, mf_item_table,
           mlp_user_table, mlp_item_table, W1, b1, W2, b2, Wp, bp):
    u = user_input.astype(jnp.int32).reshape(NW, NCHUNK, CHUNK)
    it = item_input.astype(jnp.int32).reshape(NW, NCHUNK, CHUNK)
    ustage, istage = _build_staged(mf_user_table, mlp_user_table,
                                   mf_item_table, mlp_item_table, W1, Wp)
    u_rows, i_rows = _sc_gather(u, it, ustage, istage, B, BPW, NCHUNK)
    return _tc_dense(u_rows, i_rows, b1, W2, b2, Wp, bp, B)
